# speedup vs baseline: 1.1641x; 1.1641x over previous
import jax
import jax.numpy as jnp
from jax import lax
from jax.experimental import pallas as pl
from jax.experimental.pallas import tpu as pltpu

_N = 10000
_E = 320000
_DS = 128
_DE = 16
_H = 64
_EPS = 1e-5

_EB = 6400
_NB = _E // _EB


def _silu(x):
    return x / (1.0 + jnp.exp(-x))


def _ab_body(sx_ref, rx_ref, w1s_ref, w1r_ref, a_ref, b_ref):
    a_ref[...] = jnp.dot(sx_ref[...], w1s_ref[...], preferred_element_type=jnp.float32)
    b_ref[...] = jnp.dot(rx_ref[...], w1r_ref[...], preferred_element_type=jnp.float32)


def _stage_ab(sender_x, receiver_x, w1s, w1r):
    return pl.pallas_call(
        _ab_body,
        out_shape=(
            jax.ShapeDtypeStruct((_N, _H), jnp.float32),
            jax.ShapeDtypeStruct((_N, _H), jnp.float32),
        ),
    )(sender_x, receiver_x, w1s, w1r)


def _edge_mlp_body(ga_ref, gb_ref, ea_ref, w1e_ref, b1_ref, w2_ref, b2_ref,
                   o_ref, stats_ref, sacc):
    pre = (ga_ref[...] + gb_ref[...]
           + jnp.dot(ea_ref[...], w1e_ref[...], preferred_element_type=jnp.float32)
           + b1_ref[...])
    h = _silu(pre)
    o = jnp.dot(h, w2_ref[...], preferred_element_type=jnp.float32) + b2_ref[...]
    o_ref[...] = o
    i = pl.program_id(0)

    @pl.when(i == 0)
    def _():
        sacc[0] = 0.0
        sacc[1] = 0.0

    sacc[0] += jnp.sum(o)
    sacc[1] += jnp.sum(o * o)

    @pl.when(i == pl.num_programs(0) - 1)
    def _():
        stats_ref[0] = sacc[0]
        stats_ref[1] = sacc[1]


def _stage_edge_mlp(ga, gb, edge_attr, w1e, eb1, eW2, eb2):
    return pl.pallas_call(
        _edge_mlp_body,
        grid=(_NB,),
        in_specs=[
            pl.BlockSpec((_EB, _H), lambda i: (i, 0)),
            pl.BlockSpec((_EB, _H), lambda i: (i, 0)),
            pl.BlockSpec((_EB, _DE), lambda i: (i, 0)),
            pl.BlockSpec((_DE, _H), lambda i: (0, 0)),
            pl.BlockSpec((1, _H), lambda i: (0, 0)),
            pl.BlockSpec((_H, _DE), lambda i: (0, 0)),
            pl.BlockSpec((1, _DE), lambda i: (0, 0)),
        ],
        out_specs=[
            pl.BlockSpec((_EB, _DE), lambda i: (i, 0)),
            pl.BlockSpec(memory_space=pltpu.SMEM),
        ],
        out_shape=(
            jax.ShapeDtypeStruct((_E, _DE), jnp.float32),
            jax.ShapeDtypeStruct((2,), jnp.float32),
        ),
        scratch_shapes=[pltpu.SMEM((2,), jnp.float32)],
    )(ga, gb, edge_attr, w1e, eb1, eW2, eb2)


def _edge_norm_body(stats_ref, o_ref, ea_ref, eg_ref, ebt_ref, out_ref):
    denom = 1.0 / (_E * _DE)
    mu = stats_ref[0] * denom
    var = stats_ref[1] * denom - mu * mu
    inv = lax.rsqrt(var + _EPS)
    a = eg_ref[...] * inv
    c = ebt_ref[...] - mu * a
    out_ref[...] = ea_ref[...] + o_ref[...] * a + c


def _stage_edge_norm(stats, o, edge_attr, eg, ebt):
    return pl.pallas_call(
        _edge_norm_body,
        grid=(_NB,),
        in_specs=[
            pl.BlockSpec(memory_space=pltpu.SMEM),
            pl.BlockSpec((_EB, _DE), lambda i: (i, 0)),
            pl.BlockSpec((_EB, _DE), lambda i: (i, 0)),
            pl.BlockSpec((1, _DE), lambda i: (0, 0)),
            pl.BlockSpec((1, _DE), lambda i: (0, 0)),
        ],
        out_specs=pl.BlockSpec((_EB, _DE), lambda i: (i, 0)),
        out_shape=jax.ShapeDtypeStruct((_E, _DE), jnp.float32),
    )(stats, o, edge_attr, eg, ebt)


def _node_body(stats_ref, p_ref, rx_ref, sx_ref,
               nW1r_ref, nW1e_ref, nb1_ref, nW2_ref, nb2_ref, ng_ref, nbt_ref,
               sW1_ref, sb1_ref, sW2_ref, sb2_ref, sg_ref, sbt_ref,
               eg_ref, ebt_ref,
               rout_ref, sout_ref):
    denom = 1.0 / (_E * _DE)
    mu = stats_ref[0] * denom
    var = stats_ref[1] * denom - mu * mu
    inv = lax.rsqrt(var + _EPS)
    a = eg_ref[...] * inv
    c = ebt_ref[...] - mu * a

    p = p_ref[...]
    seg = p[0:_N, 0:_DE] + p[_N:2 * _N, 0:_DE]
    cnt = p[0:_N, _DE:_DE + 1] + p[_N:2 * _N, _DE:_DE + 1]
    aggr = (seg * a + cnt * c) / jnp.maximum(cnt, 1.0)

    rx = rx_ref[...]
    pre = (jnp.dot(rx, nW1r_ref[...], preferred_element_type=jnp.float32)
           + jnp.dot(aggr, nW1e_ref[...], preferred_element_type=jnp.float32)
           + nb1_ref[...])
    h = _silu(pre)
    y = jnp.dot(h, nW2_ref[...], preferred_element_type=jnp.float32) + nb2_ref[...]
    mu_y = jnp.mean(y)
    var_y = jnp.mean((y - mu_y) ** 2)
    yn = (y - mu_y) * lax.rsqrt(var_y + _EPS) * ng_ref[...] + nbt_ref[...]
    rout_ref[...] = rx + yn

    sx = sx_ref[...]
    pre_s = jnp.dot(sx, sW1_ref[...], preferred_element_type=jnp.float32) + sb1_ref[...]
    hs = _silu(pre_s)
    ys = jnp.dot(hs, sW2_ref[...], preferred_element_type=jnp.float32) + sb2_ref[...]
    mu_s = jnp.mean(ys)
    var_s = jnp.mean((ys - mu_s) ** 2)
    ysn = (ys - mu_s) * lax.rsqrt(var_s + _EPS) * sg_ref[...] + sbt_ref[...]
    sout_ref[...] = sx + ysn


def _stage_node(stats, p, receiver_x, sender_x,
                nW1r, nW1e, nb1, nW2, nb2, ng, nbt,
                sW1, sb1, sW2, sb2, sg, sbt, eg, ebt):
    n_rest = 18
    return pl.pallas_call(
        _node_body,
        in_specs=[pl.BlockSpec(memory_space=pltpu.SMEM)]
                 + [pl.BlockSpec() for _ in range(n_rest)],
        out_shape=(
            jax.ShapeDtypeStruct((_N, _DS), jnp.float32),
            jax.ShapeDtypeStruct((_N, _DS), jnp.float32),
        ),
    )(stats, p, receiver_x, sender_x,
      nW1r, nW1e, nb1, nW2, nb2, ng, nbt,
      sW1, sb1, sW2, sb2, sg, sbt, eg, ebt)


def kernel(sender_x, receiver_x, edge_attr, edge_index,
           eW1, eb1, eW2, eb2, eg, ebt,
           nW1, nb1, nW2, nb2, ng, nbt,
           sW1, sb1, sW2, sb2, sg, sbt):
    ei0 = edge_index[0]
    ei1 = edge_index[1]

    w1s = eW1[0:_DS]
    w1r = eW1[_DS:2 * _DS]
    w1e = eW1[2 * _DS:]
    eb1_2 = eb1.reshape(1, _H)
    eb2_2 = eb2.reshape(1, _DE)
    eg_2 = eg.reshape(1, _DE)
    ebt_2 = ebt.reshape(1, _DE)
    nW1r = nW1[0:_DS]
    nW1e = nW1[_DS:]
    nb1_2 = nb1.reshape(1, _H)
    nb2_2 = nb2.reshape(1, _DS)
    ng_2 = ng.reshape(1, _DS)
    nbt_2 = nbt.reshape(1, _DS)
    sb1_2 = sb1.reshape(1, _H)
    sb2_2 = sb2.reshape(1, _DS)
    sg_2 = sg.reshape(1, _DS)
    sbt_2 = sbt.reshape(1, _DS)

    a_tab, b_tab = _stage_ab(sender_x, receiver_x, w1s, w1r)

    ga = a_tab[ei0]
    gb = b_tab[ei1]

    o, stats = _stage_edge_mlp(ga, gb, edge_attr, w1e, eb1_2, eW2, eb2_2)

    pay = jnp.concatenate(
        [o, jnp.ones((_E, 1), jnp.float32), jnp.zeros((_E, 15), jnp.float32)], axis=1)
    p0 = jax.ops.segment_sum(pay[: _E // 2], ei1[: _E // 2], num_segments=_N)
    p1 = jax.ops.segment_sum(pay[_E // 2:], ei1[_E // 2:], num_segments=_N)
    p = jnp.concatenate([p0, p1], axis=0)

    edge_out = _stage_edge_norm(stats, o, edge_attr, eg_2, ebt_2)

    receiver_out, sender_out = _stage_node(
        stats, p, receiver_x, sender_x,
        nW1r, nW1e, nb1_2, nW2, nb2_2, ng_2, nbt_2,
        sW1, sb1_2, sW2, sb2_2, sg_2, sbt_2, eg_2, ebt_2)

    return (sender_out, receiver_out, edge_out)


# speedup vs baseline: 2.0823x; 1.7888x over previous
import jax
import jax.numpy as jnp
from jax import lax
from jax.experimental import pallas as pl
from jax.experimental.pallas import tpu as pltpu
from jax.experimental.pallas import tpu_sc as plsc

_N = 10000
_E = 320000
_DS = 128
_DE = 16
_H = 64
_EPS = 1e-5

_EB = 6400
_NB = _E // _EB

_NC = 2
_NS = 16
_NW = _NC * _NS
_EPW = _E // _NW
_CH = 80
_NCHUNK = _EPW // _CH


def _sc_gather_body(t_hbm, ei0_hbm, ei1_hbm, gs_hbm,
                    t_sh, idx0_v, idx1_v, rows_t0, rows_t1, rows_s,
                    sem_a, sem_b):
    sid = lax.axis_index("s")
    wid = sid * _NC + lax.axis_index("c")
    base_w = wid * _EPW

    @pl.when(sid < 15)
    def _():
        pltpu.sync_copy(t_hbm.at[pl.ds(sid * 640, 640)], t_sh.at[pl.ds(sid * 640, 640)])

    @pl.when(sid == 15)
    def _():
        pltpu.sync_copy(t_hbm.at[pl.ds(9600, 400)], t_sh.at[pl.ds(9600, 400)])

    plsc.subcore_barrier()

    def body(i, carry):
        base = base_w + i * _CH
        pltpu.sync_copy(ei0_hbm.at[pl.ds(base, _CH)], idx0_v)
        pltpu.sync_copy(ei1_hbm.at[pl.ds(base, _CH)], idx1_v)
        cpa = pltpu.async_copy(t_sh.at[idx0_v], rows_t0, sem_a)
        cpb = pltpu.async_copy(t_sh.at[idx1_v], rows_t1, sem_b)
        cpa.wait()
        cpb.wait()

        def row_add(r, c2):
            for cc in range(4):
                rows_s[r, pl.ds(16 * cc, 16)] = (
                    rows_t0[r, pl.ds(16 * cc, 16)]
                    + rows_t1[r, pl.ds(_H + 16 * cc, 16)])
            return c2

        lax.fori_loop(0, _CH, row_add, 0)
        pltpu.sync_copy(rows_s, gs_hbm.at[pl.ds(base, _CH)])
        return carry

    lax.fori_loop(0, _NCHUNK, body, 0)


def _stage_sc_gather(t_tab, ei0, ei1):
    f = pl.kernel(
        _sc_gather_body,
        out_type=jax.ShapeDtypeStruct((_E, _H), jnp.float32),
        mesh=plsc.VectorSubcoreMesh(core_axis_name="c", subcore_axis_name="s"),
        scratch_types=[
            pltpu.VMEM_SHARED((_N, 2 * _H), jnp.float32),
            pltpu.VMEM((_CH,), jnp.int32),
            pltpu.VMEM((_CH,), jnp.int32),
            pltpu.VMEM((_CH, 2 * _H), jnp.float32),
            pltpu.VMEM((_CH, 2 * _H), jnp.float32),
            pltpu.VMEM((_CH, _H), jnp.float32),
            pltpu.SemaphoreType.DMA,
            pltpu.SemaphoreType.DMA,
        ],
    )
    return f(t_tab, ei0, ei1)


def _silu(x):
    return x / (1.0 + jnp.exp(-x))


def _ab_body(sx_ref, rx_ref, w1s_ref, w1r_ref, t_ref):
    t_ref[:, 0:_H] = jnp.dot(sx_ref[...], w1s_ref[...], preferred_element_type=jnp.float32)
    t_ref[:, _H:2 * _H] = jnp.dot(rx_ref[...], w1r_ref[...], preferred_element_type=jnp.float32)


def _stage_ab(sender_x, receiver_x, w1s, w1r):
    return pl.pallas_call(
        _ab_body,
        out_shape=jax.ShapeDtypeStruct((_N, 2 * _H), jnp.float32),
    )(sender_x, receiver_x, w1s, w1r)


def _edge_mlp_body(gs_ref, ea_ref, w1e_ref, b1_ref, w2_ref, b2_ref,
                   o_ref, stats_ref, sacc):
    pre = (gs_ref[...]
           + jnp.dot(ea_ref[...], w1e_ref[...], preferred_element_type=jnp.float32)
           + b1_ref[...])
    h = _silu(pre)
    o = jnp.dot(h, w2_ref[...], preferred_element_type=jnp.float32) + b2_ref[...]
    o_ref[...] = o
    i = pl.program_id(0)

    @pl.when(i == 0)
    def _():
        sacc[0] = 0.0
        sacc[1] = 0.0

    sacc[0] += jnp.sum(o)
    sacc[1] += jnp.sum(o * o)

    @pl.when(i == pl.num_programs(0) - 1)
    def _():
        stats_ref[0] = sacc[0]
        stats_ref[1] = sacc[1]


def _stage_edge_mlp(gs, edge_attr, w1e, eb1, eW2, eb2):
    return pl.pallas_call(
        _edge_mlp_body,
        grid=(_NB,),
        in_specs=[
            pl.BlockSpec((_EB, _H), lambda i: (i, 0)),
            pl.BlockSpec((_EB, _DE), lambda i: (i, 0)),
            pl.BlockSpec((_DE, _H), lambda i: (0, 0)),
            pl.BlockSpec((1, _H), lambda i: (0, 0)),
            pl.BlockSpec((_H, _DE), lambda i: (0, 0)),
            pl.BlockSpec((1, _DE), lambda i: (0, 0)),
        ],
        out_specs=[
            pl.BlockSpec((_EB, _DE), lambda i: (i, 0)),
            pl.BlockSpec(memory_space=pltpu.SMEM),
        ],
        out_shape=(
            jax.ShapeDtypeStruct((_E, _DE), jnp.float32),
            jax.ShapeDtypeStruct((2,), jnp.float32),
        ),
        scratch_shapes=[pltpu.SMEM((2,), jnp.float32)],
    )(gs, edge_attr, w1e, eb1, eW2, eb2)


def _edge_norm_body(stats_ref, o_ref, ea_ref, eg_ref, ebt_ref, out_ref):
    denom = 1.0 / (_E * _DE)
    mu = stats_ref[0] * denom
    var = stats_ref[1] * denom - mu * mu
    inv = lax.rsqrt(var + _EPS)
    a = eg_ref[...] * inv
    c = ebt_ref[...] - mu * a
    out_ref[...] = ea_ref[...] + o_ref[...] * a + c


def _stage_edge_norm(stats, o, edge_attr, eg, ebt):
    return pl.pallas_call(
        _edge_norm_body,
        grid=(_NB,),
        in_specs=[
            pl.BlockSpec(memory_space=pltpu.SMEM),
            pl.BlockSpec((_EB, _DE), lambda i: (i, 0)),
            pl.BlockSpec((_EB, _DE), lambda i: (i, 0)),
            pl.BlockSpec((1, _DE), lambda i: (0, 0)),
            pl.BlockSpec((1, _DE), lambda i: (0, 0)),
        ],
        out_specs=pl.BlockSpec((_EB, _DE), lambda i: (i, 0)),
        out_shape=jax.ShapeDtypeStruct((_E, _DE), jnp.float32),
    )(stats, o, edge_attr, eg, ebt)


def _node_body(stats_ref, p_ref, rx_ref, sx_ref,
               nW1r_ref, nW1e_ref, nb1_ref, nW2_ref, nb2_ref, ng_ref, nbt_ref,
               sW1_ref, sb1_ref, sW2_ref, sb2_ref, sg_ref, sbt_ref,
               eg_ref, ebt_ref,
               rout_ref, sout_ref):
    denom = 1.0 / (_E * _DE)
    mu = stats_ref[0] * denom
    var = stats_ref[1] * denom - mu * mu
    inv = lax.rsqrt(var + _EPS)
    a = eg_ref[...] * inv
    c = ebt_ref[...] - mu * a

    p = p_ref[...]
    seg = p[0:_N, 0:_DE] + p[_N:2 * _N, 0:_DE]
    cnt = p[0:_N, _DE:_DE + 1] + p[_N:2 * _N, _DE:_DE + 1]
    aggr = (seg * a + cnt * c) / jnp.maximum(cnt, 1.0)

    rx = rx_ref[...]
    pre = (jnp.dot(rx, nW1r_ref[...], preferred_element_type=jnp.float32)
           + jnp.dot(aggr, nW1e_ref[...], preferred_element_type=jnp.float32)
           + nb1_ref[...])
    h = _silu(pre)
    y = jnp.dot(h, nW2_ref[...], preferred_element_type=jnp.float32) + nb2_ref[...]
    mu_y = jnp.mean(y)
    var_y = jnp.mean((y - mu_y) ** 2)
    yn = (y - mu_y) * lax.rsqrt(var_y + _EPS) * ng_ref[...] + nbt_ref[...]
    rout_ref[...] = rx + yn

    sx = sx_ref[...]
    pre_s = jnp.dot(sx, sW1_ref[...], preferred_element_type=jnp.float32) + sb1_ref[...]
    hs = _silu(pre_s)
    ys = jnp.dot(hs, sW2_ref[...], preferred_element_type=jnp.float32) + sb2_ref[...]
    mu_s = jnp.mean(ys)
    var_s = jnp.mean((ys - mu_s) ** 2)
    ysn = (ys - mu_s) * lax.rsqrt(var_s + _EPS) * sg_ref[...] + sbt_ref[...]
    sout_ref[...] = sx + ysn


def _stage_node(stats, p, receiver_x, sender_x,
                nW1r, nW1e, nb1, nW2, nb2, ng, nbt,
                sW1, sb1, sW2, sb2, sg, sbt, eg, ebt):
    n_rest = 18
    return pl.pallas_call(
        _node_body,
        in_specs=[pl.BlockSpec(memory_space=pltpu.SMEM)]
                 + [pl.BlockSpec() for _ in range(n_rest)],
        out_shape=(
            jax.ShapeDtypeStruct((_N, _DS), jnp.float32),
            jax.ShapeDtypeStruct((_N, _DS), jnp.float32),
        ),
    )(stats, p, receiver_x, sender_x,
      nW1r, nW1e, nb1, nW2, nb2, ng, nbt,
      sW1, sb1, sW2, sb2, sg, sbt, eg, ebt)


def kernel(sender_x, receiver_x, edge_attr, edge_index,
           eW1, eb1, eW2, eb2, eg, ebt,
           nW1, nb1, nW2, nb2, ng, nbt,
           sW1, sb1, sW2, sb2, sg, sbt):
    ei0 = edge_index[0]
    ei1 = edge_index[1]

    w1s = eW1[0:_DS]
    w1r = eW1[_DS:2 * _DS]
    w1e = eW1[2 * _DS:]
    eb1_2 = eb1.reshape(1, _H)
    eb2_2 = eb2.reshape(1, _DE)
    eg_2 = eg.reshape(1, _DE)
    ebt_2 = ebt.reshape(1, _DE)
    nW1r = nW1[0:_DS]
    nW1e = nW1[_DS:]
    nb1_2 = nb1.reshape(1, _H)
    nb2_2 = nb2.reshape(1, _DS)
    ng_2 = ng.reshape(1, _DS)
    nbt_2 = nbt.reshape(1, _DS)
    sb1_2 = sb1.reshape(1, _H)
    sb2_2 = sb2.reshape(1, _DS)
    sg_2 = sg.reshape(1, _DS)
    sbt_2 = sbt.reshape(1, _DS)

    t_tab = _stage_ab(sender_x, receiver_x, w1s, w1r)

    gs = _stage_sc_gather(t_tab, ei0, ei1)

    o, stats = _stage_edge_mlp(gs, edge_attr, w1e, eb1_2, eW2, eb2_2)

    pay = jnp.concatenate(
        [o, jnp.ones((_E, 1), jnp.float32), jnp.zeros((_E, 15), jnp.float32)], axis=1)
    p0 = jax.ops.segment_sum(pay[: _E // 2], ei1[: _E // 2], num_segments=_N)
    p1 = jax.ops.segment_sum(pay[_E // 2:], ei1[_E // 2:], num_segments=_N)
    p = jnp.concatenate([p0, p1], axis=0)

    edge_out = _stage_edge_norm(stats, o, edge_attr, eg_2, ebt_2)

    receiver_out, sender_out = _stage_node(
        stats, p, receiver_x, sender_x,
        nW1r, nW1e, nb1_2, nW2, nb2_2, ng_2, nbt_2,
        sW1, sb1_2, sW2, sb2_2, sg_2, sbt_2, eg_2, ebt_2)

    return (sender_out, receiver_out, edge_out)


# speedup vs baseline: 3.2763x; 1.5734x over previous
import jax
import jax.numpy as jnp
from jax import lax
from jax.experimental import pallas as pl
from jax.experimental.pallas import tpu as pltpu
from jax.experimental.pallas import tpu_sc as plsc

_N = 10000
_E = 320000
_DS = 128
_DE = 16
_H = 64
_EPS = 1e-5

_EB = 6400
_NB = _E // _EB

_NC = 2
_NS = 16
_NW = _NC * _NS
_EPW = _E // _NW
_CH = 80
_NCHUNK = _EPW // _CH


def _sc_gather_body(t_hbm, ei0_hbm, ei1_hbm, gs_hbm,
                    t_sh, idx0_v, idx1_v, rows_t0, rows_t1, rows_s,
                    sem_a, sem_b):
    sid = lax.axis_index("s")
    wid = sid * _NC + lax.axis_index("c")
    base_w = wid * _EPW

    @pl.when(sid < 15)
    def _():
        pltpu.sync_copy(t_hbm.at[pl.ds(sid * 640, 640)], t_sh.at[pl.ds(sid * 640, 640)])

    @pl.when(sid == 15)
    def _():
        pltpu.sync_copy(t_hbm.at[pl.ds(9600, 400)], t_sh.at[pl.ds(9600, 400)])

    plsc.subcore_barrier()

    def body(i, carry):
        base = base_w + i * _CH
        pltpu.sync_copy(ei0_hbm.at[pl.ds(base, _CH)], idx0_v)
        pltpu.sync_copy(ei1_hbm.at[pl.ds(base, _CH)], idx1_v)
        cpa = pltpu.async_copy(t_sh.at[idx0_v], rows_t0, sem_a)
        cpb = pltpu.async_copy(t_sh.at[idx1_v], rows_t1, sem_b)
        cpa.wait()
        cpb.wait()

        def row_add(r, c2):
            for cc in range(4):
                rows_s[r, pl.ds(16 * cc, 16)] = (
                    rows_t0[r, pl.ds(16 * cc, 16)]
                    + rows_t1[r, pl.ds(_H + 16 * cc, 16)])
            return c2

        lax.fori_loop(0, _CH, row_add, 0)
        pltpu.sync_copy(rows_s, gs_hbm.at[pl.ds(base, _CH)])
        return carry

    lax.fori_loop(0, _NCHUNK, body, 0)


def _stage_sc_gather(t_tab, ei0, ei1):
    f = pl.kernel(
        _sc_gather_body,
        out_type=jax.ShapeDtypeStruct((_E, _H), jnp.float32),
        mesh=plsc.VectorSubcoreMesh(core_axis_name="c", subcore_axis_name="s"),
        scratch_types=[
            pltpu.VMEM_SHARED((_N, 2 * _H), jnp.float32),
            pltpu.VMEM((_CH,), jnp.int32),
            pltpu.VMEM((_CH,), jnp.int32),
            pltpu.VMEM((_CH, 2 * _H), jnp.float32),
            pltpu.VMEM((_CH, 2 * _H), jnp.float32),
            pltpu.VMEM((_CH, _H), jnp.float32),
            pltpu.SemaphoreType.DMA,
            pltpu.SemaphoreType.DMA,
        ],
    )
    return f(t_tab, ei0, ei1)


def _silu(x):
    return x / (1.0 + jnp.exp(-x))


def _ab_body(sx_ref, rx_ref, w1s_ref, w1r_ref, t_ref):
    t_ref[:, 0:_H] = jnp.dot(sx_ref[...], w1s_ref[...], preferred_element_type=jnp.float32)
    t_ref[:, _H:2 * _H] = jnp.dot(rx_ref[...], w1r_ref[...], preferred_element_type=jnp.float32)


def _stage_ab(sender_x, receiver_x, w1s, w1r):
    return pl.pallas_call(
        _ab_body,
        out_shape=jax.ShapeDtypeStruct((_N, 2 * _H), jnp.float32),
    )(sender_x, receiver_x, w1s, w1r)


def _edge_mlp_body(gs_ref, ea_ref, w1e_ref, b1_ref, w2_ref, b2_ref,
                   o_ref, stats_ref, sacc):
    pre = (gs_ref[...]
           + jnp.dot(ea_ref[...], w1e_ref[...], preferred_element_type=jnp.float32)
           + b1_ref[...])
    h = _silu(pre)
    o = jnp.dot(h, w2_ref[...], preferred_element_type=jnp.float32) + b2_ref[...]
    o_ref[...] = o
    i = pl.program_id(0)

    @pl.when(i == 0)
    def _():
        sacc[0] = 0.0
        sacc[1] = 0.0

    sacc[0] += jnp.sum(o)
    sacc[1] += jnp.sum(o * o)

    @pl.when(i == pl.num_programs(0) - 1)
    def _():
        stats_ref[0] = sacc[0]
        stats_ref[1] = sacc[1]


def _stage_edge_mlp(gs, edge_attr, w1e, eb1, eW2, eb2):
    return pl.pallas_call(
        _edge_mlp_body,
        grid=(_NB,),
        in_specs=[
            pl.BlockSpec((_EB, _H), lambda i: (i, 0)),
            pl.BlockSpec((_EB, _DE), lambda i: (i, 0)),
            pl.BlockSpec((_DE, _H), lambda i: (0, 0)),
            pl.BlockSpec((1, _H), lambda i: (0, 0)),
            pl.BlockSpec((_H, _DE), lambda i: (0, 0)),
            pl.BlockSpec((1, _DE), lambda i: (0, 0)),
        ],
        out_specs=[
            pl.BlockSpec((_EB, _DE), lambda i: (i, 0)),
            pl.BlockSpec(memory_space=pltpu.SMEM),
        ],
        out_shape=(
            jax.ShapeDtypeStruct((_E, _DE), jnp.float32),
            jax.ShapeDtypeStruct((2,), jnp.float32),
        ),
        scratch_shapes=[pltpu.SMEM((2,), jnp.float32)],
    )(gs, edge_attr, w1e, eb1, eW2, eb2)


_SCCH = 128
_SEPW = 10240
_WB = 640


def _sc_scatter_body(o2_hbm, ei1_hbm, p_hbm,
                     acc, obuf, pay, idx_v, zbuf, sem):
    sid = lax.axis_index("s")
    cid = lax.axis_index("c")
    wid = sid * _NC + cid

    def zrow(r, c2):
        for s in range(8):
            zbuf[r, pl.ds(16 * s, 16)] = jnp.zeros((16,), jnp.float32)
        return c2

    lax.fori_loop(0, 80, zrow, 0)

    def zcp(j, c2):
        pltpu.sync_copy(zbuf, acc.at[pl.ds(sid * 640 + j * 80, 80)])
        return c2

    @pl.when(sid < 15)
    def _():
        lax.fori_loop(0, 8, zcp, 0)

    @pl.when(sid == 15)
    def _():
        lax.fori_loop(0, 5, zcp, 0)

    ones0 = jnp.where(lax.iota(jnp.int32, 16) == 0,
                      jnp.float32(1.0), jnp.float32(0.0))

    def crow(r, c2):
        pay[r, pl.ds(16, 16)] = ones0
        for s in range(2, 8):
            pay[r, pl.ds(16 * s, 16)] = jnp.zeros((16,), jnp.float32)
        return c2

    lax.fori_loop(0, _SCCH, crow, 0)

    plsc.subcore_barrier()

    ebase_w = wid * _SEPW
    rbase_w = wid * (_SEPW // 8)

    def chunk(i, c2):
        ebase = ebase_w + i * _SCCH
        pltpu.sync_copy(ei1_hbm.at[pl.ds(ebase, _SCCH)], idx_v)
        pltpu.sync_copy(o2_hbm.at[pl.ds(rbase_w + i * (_SCCH // 8), _SCCH // 8)], obuf)

        def prow(r, c3):
            for s in range(8):
                pay[8 * r + s, pl.ds(0, 16)] = obuf[r, pl.ds(16 * s, 16)]
            return c3

        lax.fori_loop(0, _SCCH // 8, prow, 0)
        pltpu.sync_copy(pay, acc.at[idx_v], add=True)
        return c2

    @pl.when(wid < 31)
    def _():
        lax.fori_loop(0, _SEPW // _SCCH, chunk, 0)

    @pl.when(wid == 31)
    def _():
        lax.fori_loop(0, (_E - 31 * _SEPW) // _SCCH, chunk, 0)

    plsc.subcore_barrier()

    @pl.when(sid < 15)
    def _():
        pltpu.sync_copy(acc.at[pl.ds(sid * 640, 640)],
                        p_hbm.at[cid, pl.ds(sid * 640, 640)])

    @pl.when(sid == 15)
    def _():
        pltpu.sync_copy(acc.at[pl.ds(9600, 400)],
                        p_hbm.at[cid, pl.ds(9600, 400)])


def _stage_sc_scatter(o2, ei1):
    f = pl.kernel(
        _sc_scatter_body,
        out_type=jax.ShapeDtypeStruct((_NC, _N, 128), jnp.float32),
        mesh=plsc.VectorSubcoreMesh(core_axis_name="c", subcore_axis_name="s"),
        scratch_types=[
            pltpu.VMEM_SHARED((_N, 128), jnp.float32),
            pltpu.VMEM((_SCCH // 8, 128), jnp.float32),
            pltpu.VMEM((_SCCH, 128), jnp.float32),
            pltpu.VMEM((_SCCH,), jnp.int32),
            pltpu.VMEM((80, 128), jnp.float32),
            pltpu.SemaphoreType.DMA,
        ],
    )
    return f(o2, ei1)


def _edge_norm_body(stats_ref, o_ref, ea_ref, eg_ref, ebt_ref, out_ref):
    denom = 1.0 / (_E * _DE)
    mu = stats_ref[0] * denom
    var = stats_ref[1] * denom - mu * mu
    inv = lax.rsqrt(var + _EPS)
    a = eg_ref[...] * inv
    c = ebt_ref[...] - mu * a
    out_ref[...] = ea_ref[...] + o_ref[...] * a + c


def _stage_edge_norm(stats, o, edge_attr, eg, ebt):
    return pl.pallas_call(
        _edge_norm_body,
        grid=(_NB,),
        in_specs=[
            pl.BlockSpec(memory_space=pltpu.SMEM),
            pl.BlockSpec((_EB, _DE), lambda i: (i, 0)),
            pl.BlockSpec((_EB, _DE), lambda i: (i, 0)),
            pl.BlockSpec((1, _DE), lambda i: (0, 0)),
            pl.BlockSpec((1, _DE), lambda i: (0, 0)),
        ],
        out_specs=pl.BlockSpec((_EB, _DE), lambda i: (i, 0)),
        out_shape=jax.ShapeDtypeStruct((_E, _DE), jnp.float32),
    )(stats, o, edge_attr, eg, ebt)


def _node_body(stats_ref, p_ref, rx_ref, sx_ref,
               nW1r_ref, nW1e_ref, nb1_ref, nW2_ref, nb2_ref, ng_ref, nbt_ref,
               sW1_ref, sb1_ref, sW2_ref, sb2_ref, sg_ref, sbt_ref,
               eg_ref, ebt_ref,
               rout_ref, sout_ref):
    denom = 1.0 / (_E * _DE)
    mu = stats_ref[0] * denom
    var = stats_ref[1] * denom - mu * mu
    inv = lax.rsqrt(var + _EPS)
    a = eg_ref[...] * inv
    c = ebt_ref[...] - mu * a

    p = p_ref[...]
    seg = p[0:_N, 0:_DE] + p[_N:2 * _N, 0:_DE]
    cnt = p[0:_N, _DE:_DE + 1] + p[_N:2 * _N, _DE:_DE + 1]
    aggr = (seg * a + cnt * c) / jnp.maximum(cnt, 1.0)

    rx = rx_ref[...]
    pre = (jnp.dot(rx, nW1r_ref[...], preferred_element_type=jnp.float32)
           + jnp.dot(aggr, nW1e_ref[...], preferred_element_type=jnp.float32)
           + nb1_ref[...])
    h = _silu(pre)
    y = jnp.dot(h, nW2_ref[...], preferred_element_type=jnp.float32) + nb2_ref[...]
    mu_y = jnp.mean(y)
    var_y = jnp.mean((y - mu_y) ** 2)
    yn = (y - mu_y) * lax.rsqrt(var_y + _EPS) * ng_ref[...] + nbt_ref[...]
    rout_ref[...] = rx + yn

    sx = sx_ref[...]
    pre_s = jnp.dot(sx, sW1_ref[...], preferred_element_type=jnp.float32) + sb1_ref[...]
    hs = _silu(pre_s)
    ys = jnp.dot(hs, sW2_ref[...], preferred_element_type=jnp.float32) + sb2_ref[...]
    mu_s = jnp.mean(ys)
    var_s = jnp.mean((ys - mu_s) ** 2)
    ysn = (ys - mu_s) * lax.rsqrt(var_s + _EPS) * sg_ref[...] + sbt_ref[...]
    sout_ref[...] = sx + ysn


def _stage_node(stats, p, receiver_x, sender_x,
                nW1r, nW1e, nb1, nW2, nb2, ng, nbt,
                sW1, sb1, sW2, sb2, sg, sbt, eg, ebt):
    n_rest = 18
    return pl.pallas_call(
        _node_body,
        in_specs=[pl.BlockSpec(memory_space=pltpu.SMEM)]
                 + [pl.BlockSpec() for _ in range(n_rest)],
        out_shape=(
            jax.ShapeDtypeStruct((_N, _DS), jnp.float32),
            jax.ShapeDtypeStruct((_N, _DS), jnp.float32),
        ),
    )(stats, p, receiver_x, sender_x,
      nW1r, nW1e, nb1, nW2, nb2, ng, nbt,
      sW1, sb1, sW2, sb2, sg, sbt, eg, ebt)


def kernel(sender_x, receiver_x, edge_attr, edge_index,
           eW1, eb1, eW2, eb2, eg, ebt,
           nW1, nb1, nW2, nb2, ng, nbt,
           sW1, sb1, sW2, sb2, sg, sbt):
    ei0 = edge_index[0]
    ei1 = edge_index[1]

    w1s = eW1[0:_DS]
    w1r = eW1[_DS:2 * _DS]
    w1e = eW1[2 * _DS:]
    eb1_2 = eb1.reshape(1, _H)
    eb2_2 = eb2.reshape(1, _DE)
    eg_2 = eg.reshape(1, _DE)
    ebt_2 = ebt.reshape(1, _DE)
    nW1r = nW1[0:_DS]
    nW1e = nW1[_DS:]
    nb1_2 = nb1.reshape(1, _H)
    nb2_2 = nb2.reshape(1, _DS)
    ng_2 = ng.reshape(1, _DS)
    nbt_2 = nbt.reshape(1, _DS)
    sb1_2 = sb1.reshape(1, _H)
    sb2_2 = sb2.reshape(1, _DS)
    sg_2 = sg.reshape(1, _DS)
    sbt_2 = sbt.reshape(1, _DS)

    t_tab = _stage_ab(sender_x, receiver_x, w1s, w1r)

    gs = _stage_sc_gather(t_tab, ei0, ei1)

    o, stats = _stage_edge_mlp(gs, edge_attr, w1e, eb1_2, eW2, eb2_2)

    o2 = o.reshape(_E // 8, 128)
    p = _stage_sc_scatter(o2, ei1).reshape(2 * _N, 128)

    edge_out = _stage_edge_norm(stats, o, edge_attr, eg_2, ebt_2)

    receiver_out, sender_out = _stage_node(
        stats, p, receiver_x, sender_x,
        nW1r, nW1e, nb1_2, nW2, nb2_2, ng_2, nbt_2,
        sW1, sb1_2, sW2, sb2_2, sg_2, sbt_2, eg_2, ebt_2)

    return (sender_out, receiver_out, edge_out)


# speedup vs baseline: 3.7669x; 1.1497x over previous
import jax
import jax.numpy as jnp
from jax import lax
from jax.experimental import pallas as pl
from jax.experimental.pallas import tpu as pltpu
from jax.experimental.pallas import tpu_sc as plsc

_N = 10000
_E = 320000
_DS = 128
_DE = 16
_H = 64
_EPS = 1e-5

_EB = 6400
_NB = _E // _EB

_NC = 2
_NS = 16
_NW = _NC * _NS
_EPW = _E // _NW
_CH = 80
_NCHUNK = _EPW // _CH


_GCH = 64
_GEPW = 10240


def _sc_gather_body(t_hbm, ei0_hbm, ei1_hbm, gs_hbm,
                    t_sh, idx0, idx1, rt0, rt1, rs,
                    gsem0, gsem1, wsem):
    sid = lax.axis_index("s")
    wid = sid * _NC + lax.axis_index("c")

    @pl.when(sid < 15)
    def _():
        pltpu.sync_copy(t_hbm.at[pl.ds(sid * 640, 640)], t_sh.at[pl.ds(sid * 640, 640)])

    @pl.when(sid == 15)
    def _():
        pltpu.sync_copy(t_hbm.at[pl.ds(9600, 400)], t_sh.at[pl.ds(9600, 400)])

    plsc.subcore_barrier()

    ebase_w = wid * _GEPW
    nch = jnp.where(wid < 31, _GEPW // _GCH, (_E - 31 * _GEPW) // _GCH)
    gsems = (gsem0, gsem1)

    def fire(j, b):
        base = ebase_w + j * _GCH
        pltpu.sync_copy(ei0_hbm.at[pl.ds(base, _GCH)], idx0.at[b])
        pltpu.sync_copy(ei1_hbm.at[pl.ds(base, _GCH)], idx1.at[b])
        pltpu.async_copy(t_sh.at[idx0.at[b]], rt0.at[b], gsems[b])
        pltpu.async_copy(t_sh.at[idx1.at[b]], rt1.at[b], gsems[b])

    fire(0, 0)

    def pair(i, carry):
        for b in range(2):
            j = 2 * i + b
            nb = 1 - b

            @pl.when(j + 1 < nch)
            def _():
                fire(j + 1, nb)

            pltpu.make_async_copy(t_sh.at[idx0.at[b]], rt0.at[b], gsems[b]).wait()
            pltpu.make_async_copy(t_sh.at[idx1.at[b]], rt1.at[b], gsems[b]).wait()

            @pl.when(j >= 1)
            def _():
                pltpu.make_async_copy(
                    rs, gs_hbm.at[pl.ds(ebase_w + (j - 1) * _GCH, _GCH)],
                    wsem).wait()

            def row_add(r, c2):
                for cc in range(4):
                    rs[r, pl.ds(16 * cc, 16)] = (
                        rt0[b, r, pl.ds(16 * cc, 16)]
                        + rt1[b, r, pl.ds(_H + 16 * cc, 16)])
                return c2

            lax.fori_loop(0, _GCH, row_add, 0)
            pltpu.async_copy(rs,
                             gs_hbm.at[pl.ds(ebase_w + j * _GCH, _GCH)],
                             wsem)
        return carry

    lax.fori_loop(0, nch // 2, pair, 0)
    pltpu.make_async_copy(rs, gs_hbm.at[pl.ds(ebase_w + (nch - 1) * _GCH, _GCH)],
                          wsem).wait()


def _stage_sc_gather(t_tab, ei0, ei1):
    f = pl.kernel(
        _sc_gather_body,
        out_type=jax.ShapeDtypeStruct((_E, _H), jnp.float32),
        mesh=plsc.VectorSubcoreMesh(core_axis_name="c", subcore_axis_name="s"),
        scratch_types=[
            pltpu.VMEM_SHARED((_N, 2 * _H), jnp.float32),
            pltpu.VMEM((2, _GCH), jnp.int32),
            pltpu.VMEM((2, _GCH), jnp.int32),
            pltpu.VMEM((2, _GCH, 2 * _H), jnp.float32),
            pltpu.VMEM((2, _GCH, 2 * _H), jnp.float32),
            pltpu.VMEM((_GCH, _H), jnp.float32),
            pltpu.SemaphoreType.DMA,
            pltpu.SemaphoreType.DMA,
            pltpu.SemaphoreType.DMA,
        ],
    )
    return f(t_tab, ei0, ei1)


def _silu(x):
    return x / (1.0 + jnp.exp(-x))


def _ab_body(sx_ref, rx_ref, w1s_ref, w1r_ref, t_ref):
    t_ref[:, 0:_H] = jnp.dot(sx_ref[...], w1s_ref[...], preferred_element_type=jnp.float32)
    t_ref[:, _H:2 * _H] = jnp.dot(rx_ref[...], w1r_ref[...], preferred_element_type=jnp.float32)


def _stage_ab(sender_x, receiver_x, w1s, w1r):
    return pl.pallas_call(
        _ab_body,
        out_shape=jax.ShapeDtypeStruct((_N, 2 * _H), jnp.float32),
    )(sender_x, receiver_x, w1s, w1r)


def _edge_mlp_body(gs_ref, ea_ref, w1e_ref, b1_ref, w2_ref, b2_ref,
                   o_ref, stats_ref, sacc):
    pre = (gs_ref[...]
           + jnp.dot(ea_ref[...], w1e_ref[...], preferred_element_type=jnp.float32)
           + b1_ref[...])
    h = _silu(pre)
    o = jnp.dot(h, w2_ref[...], preferred_element_type=jnp.float32) + b2_ref[...]
    o_ref[...] = o
    i = pl.program_id(0)

    @pl.when(i == 0)
    def _():
        sacc[0] = 0.0
        sacc[1] = 0.0

    sacc[0] += jnp.sum(o)
    sacc[1] += jnp.sum(o * o)

    @pl.when(i == pl.num_programs(0) - 1)
    def _():
        stats_ref[0] = sacc[0]
        stats_ref[1] = sacc[1]


def _stage_edge_mlp(gs, edge_attr, w1e, eb1, eW2, eb2):
    return pl.pallas_call(
        _edge_mlp_body,
        grid=(_NB,),
        in_specs=[
            pl.BlockSpec((_EB, _H), lambda i: (i, 0)),
            pl.BlockSpec((_EB, _DE), lambda i: (i, 0)),
            pl.BlockSpec((_DE, _H), lambda i: (0, 0)),
            pl.BlockSpec((1, _H), lambda i: (0, 0)),
            pl.BlockSpec((_H, _DE), lambda i: (0, 0)),
            pl.BlockSpec((1, _DE), lambda i: (0, 0)),
        ],
        out_specs=[
            pl.BlockSpec((_EB, _DE), lambda i: (i, 0)),
            pl.BlockSpec(memory_space=pltpu.SMEM),
        ],
        out_shape=(
            jax.ShapeDtypeStruct((_E, _DE), jnp.float32),
            jax.ShapeDtypeStruct((2,), jnp.float32),
        ),
        scratch_shapes=[pltpu.SMEM((2,), jnp.float32)],
    )(gs, edge_attr, w1e, eb1, eW2, eb2)


_SCCH = 128
_SEPW = 10240
_WB = 640


def _sc_scatter_body(o2_hbm, ei1_hbm, p_hbm,
                     acc, obuf, pay, idx_v, zbuf, sem):
    sid = lax.axis_index("s")
    cid = lax.axis_index("c")
    wid = sid * _NC + cid

    def zrow(r, c2):
        for s in range(8):
            zbuf[r, pl.ds(16 * s, 16)] = jnp.zeros((16,), jnp.float32)
        return c2

    lax.fori_loop(0, 80, zrow, 0)

    def zcp(j, c2):
        pltpu.sync_copy(zbuf, acc.at[pl.ds(sid * 640 + j * 80, 80)])
        return c2

    @pl.when(sid < 15)
    def _():
        lax.fori_loop(0, 8, zcp, 0)

    @pl.when(sid == 15)
    def _():
        lax.fori_loop(0, 5, zcp, 0)

    ones0 = jnp.where(lax.iota(jnp.int32, 16) == 0,
                      jnp.float32(1.0), jnp.float32(0.0))

    def crow(r, c2):
        pay[r, pl.ds(16, 16)] = ones0
        for s in range(2, 8):
            pay[r, pl.ds(16 * s, 16)] = jnp.zeros((16,), jnp.float32)
        return c2

    lax.fori_loop(0, _SCCH, crow, 0)

    plsc.subcore_barrier()

    ebase_w = wid * _SEPW
    rbase_w = wid * (_SEPW // 8)

    def chunk(i, c2):
        ebase = ebase_w + i * _SCCH
        pltpu.sync_copy(ei1_hbm.at[pl.ds(ebase, _SCCH)], idx_v)
        pltpu.sync_copy(o2_hbm.at[pl.ds(rbase_w + i * (_SCCH // 8), _SCCH // 8)], obuf)

        def prow(r, c3):
            for s in range(8):
                pay[8 * r + s, pl.ds(0, 16)] = obuf[r, pl.ds(16 * s, 16)]
            return c3

        lax.fori_loop(0, _SCCH // 8, prow, 0)
        pltpu.sync_copy(pay, acc.at[idx_v], add=True)
        return c2

    @pl.when(wid < 31)
    def _():
        lax.fori_loop(0, _SEPW // _SCCH, chunk, 0)

    @pl.when(wid == 31)
    def _():
        lax.fori_loop(0, (_E - 31 * _SEPW) // _SCCH, chunk, 0)

    plsc.subcore_barrier()

    @pl.when(sid < 15)
    def _():
        pltpu.sync_copy(acc.at[pl.ds(sid * 640, 640)],
                        p_hbm.at[cid, pl.ds(sid * 640, 640)])

    @pl.when(sid == 15)
    def _():
        pltpu.sync_copy(acc.at[pl.ds(9600, 400)],
                        p_hbm.at[cid, pl.ds(9600, 400)])


def _stage_sc_scatter(o2, ei1):
    f = pl.kernel(
        _sc_scatter_body,
        out_type=jax.ShapeDtypeStruct((_NC, _N, 128), jnp.float32),
        mesh=plsc.VectorSubcoreMesh(core_axis_name="c", subcore_axis_name="s"),
        scratch_types=[
            pltpu.VMEM_SHARED((_N, 128), jnp.float32),
            pltpu.VMEM((_SCCH // 8, 128), jnp.float32),
            pltpu.VMEM((_SCCH, 128), jnp.float32),
            pltpu.VMEM((_SCCH,), jnp.int32),
            pltpu.VMEM((80, 128), jnp.float32),
            pltpu.SemaphoreType.DMA,
        ],
    )
    return f(o2, ei1)


def _edge_norm_body(stats_ref, o_ref, ea_ref, eg_ref, ebt_ref, out_ref):
    denom = 1.0 / (_E * _DE)
    mu = stats_ref[0] * denom
    var = stats_ref[1] * denom - mu * mu
    inv = lax.rsqrt(var + _EPS)
    a = eg_ref[...] * inv
    c = ebt_ref[...] - mu * a
    out_ref[...] = ea_ref[...] + o_ref[...] * a + c


def _stage_edge_norm(stats, o, edge_attr, eg, ebt):
    return pl.pallas_call(
        _edge_norm_body,
        grid=(_NB,),
        in_specs=[
            pl.BlockSpec(memory_space=pltpu.SMEM),
            pl.BlockSpec((_EB, _DE), lambda i: (i, 0)),
            pl.BlockSpec((_EB, _DE), lambda i: (i, 0)),
            pl.BlockSpec((1, _DE), lambda i: (0, 0)),
            pl.BlockSpec((1, _DE), lambda i: (0, 0)),
        ],
        out_specs=pl.BlockSpec((_EB, _DE), lambda i: (i, 0)),
        out_shape=jax.ShapeDtypeStruct((_E, _DE), jnp.float32),
    )(stats, o, edge_attr, eg, ebt)


def _node_body(stats_ref, p_ref, rx_ref, sx_ref,
               nW1r_ref, nW1e_ref, nb1_ref, nW2_ref, nb2_ref, ng_ref, nbt_ref,
               sW1_ref, sb1_ref, sW2_ref, sb2_ref, sg_ref, sbt_ref,
               eg_ref, ebt_ref,
               rout_ref, sout_ref):
    denom = 1.0 / (_E * _DE)
    mu = stats_ref[0] * denom
    var = stats_ref[1] * denom - mu * mu
    inv = lax.rsqrt(var + _EPS)
    a = eg_ref[...] * inv
    c = ebt_ref[...] - mu * a

    p = p_ref[...]
    seg = p[0:_N, 0:_DE] + p[_N:2 * _N, 0:_DE]
    cnt = p[0:_N, _DE:_DE + 1] + p[_N:2 * _N, _DE:_DE + 1]
    aggr = (seg * a + cnt * c) / jnp.maximum(cnt, 1.0)

    rx = rx_ref[...]
    pre = (jnp.dot(rx, nW1r_ref[...], preferred_element_type=jnp.float32)
           + jnp.dot(aggr, nW1e_ref[...], preferred_element_type=jnp.float32)
           + nb1_ref[...])
    h = _silu(pre)
    y = jnp.dot(h, nW2_ref[...], preferred_element_type=jnp.float32) + nb2_ref[...]
    mu_y = jnp.mean(y)
    var_y = jnp.mean((y - mu_y) ** 2)
    yn = (y - mu_y) * lax.rsqrt(var_y + _EPS) * ng_ref[...] + nbt_ref[...]
    rout_ref[...] = rx + yn

    sx = sx_ref[...]
    pre_s = jnp.dot(sx, sW1_ref[...], preferred_element_type=jnp.float32) + sb1_ref[...]
    hs = _silu(pre_s)
    ys = jnp.dot(hs, sW2_ref[...], preferred_element_type=jnp.float32) + sb2_ref[...]
    mu_s = jnp.mean(ys)
    var_s = jnp.mean((ys - mu_s) ** 2)
    ysn = (ys - mu_s) * lax.rsqrt(var_s + _EPS) * sg_ref[...] + sbt_ref[...]
    sout_ref[...] = sx + ysn


def _stage_node(stats, p, receiver_x, sender_x,
                nW1r, nW1e, nb1, nW2, nb2, ng, nbt,
                sW1, sb1, sW2, sb2, sg, sbt, eg, ebt):
    n_rest = 18
    return pl.pallas_call(
        _node_body,
        in_specs=[pl.BlockSpec(memory_space=pltpu.SMEM)]
                 + [pl.BlockSpec() for _ in range(n_rest)],
        out_shape=(
            jax.ShapeDtypeStruct((_N, _DS), jnp.float32),
            jax.ShapeDtypeStruct((_N, _DS), jnp.float32),
        ),
    )(stats, p, receiver_x, sender_x,
      nW1r, nW1e, nb1, nW2, nb2, ng, nbt,
      sW1, sb1, sW2, sb2, sg, sbt, eg, ebt)


def kernel(sender_x, receiver_x, edge_attr, edge_index,
           eW1, eb1, eW2, eb2, eg, ebt,
           nW1, nb1, nW2, nb2, ng, nbt,
           sW1, sb1, sW2, sb2, sg, sbt):
    ei0 = edge_index[0]
    ei1 = edge_index[1]

    w1s = eW1[0:_DS]
    w1r = eW1[_DS:2 * _DS]
    w1e = eW1[2 * _DS:]
    eb1_2 = eb1.reshape(1, _H)
    eb2_2 = eb2.reshape(1, _DE)
    eg_2 = eg.reshape(1, _DE)
    ebt_2 = ebt.reshape(1, _DE)
    nW1r = nW1[0:_DS]
    nW1e = nW1[_DS:]
    nb1_2 = nb1.reshape(1, _H)
    nb2_2 = nb2.reshape(1, _DS)
    ng_2 = ng.reshape(1, _DS)
    nbt_2 = nbt.reshape(1, _DS)
    sb1_2 = sb1.reshape(1, _H)
    sb2_2 = sb2.reshape(1, _DS)
    sg_2 = sg.reshape(1, _DS)
    sbt_2 = sbt.reshape(1, _DS)

    t_tab = _stage_ab(sender_x, receiver_x, w1s, w1r)

    gs = _stage_sc_gather(t_tab, ei0, ei1)

    o, stats = _stage_edge_mlp(gs, edge_attr, w1e, eb1_2, eW2, eb2_2)

    o2 = o.reshape(_E // 8, 128)
    p = _stage_sc_scatter(o2, ei1).reshape(2 * _N, 128)

    edge_out = _stage_edge_norm(stats, o, edge_attr, eg_2, ebt_2)

    receiver_out, sender_out = _stage_node(
        stats, p, receiver_x, sender_x,
        nW1r, nW1e, nb1_2, nW2, nb2_2, ng_2, nbt_2,
        sW1, sb1_2, sW2, sb2_2, sg_2, sbt_2, eg_2, ebt_2)

    return (sender_out, receiver_out, edge_out)


# speedup vs baseline: 3.8454x; 1.0209x over previous
import jax
import jax.numpy as jnp
from jax import lax
from jax.experimental import pallas as pl
from jax.experimental.pallas import tpu as pltpu
from jax.experimental.pallas import tpu_sc as plsc

_N = 10000
_E = 320000
_DS = 128
_DE = 16
_H = 64
_EPS = 1e-5

_EB = 6400
_NB = _E // _EB

_NC = 2
_NS = 16
_NW = _NC * _NS
_EPW = _E // _NW
_CH = 80
_NCHUNK = _EPW // _CH


_GCH = 64
_GEPW = 10240


def _sc_gather_body(t_hbm, ei0_hbm, ei1_hbm, gs_hbm,
                    t_sh, idx0, idx1, rt0, rt1, rs,
                    gsem0, gsem1, wsem):
    sid = lax.axis_index("s")
    wid = sid * _NC + lax.axis_index("c")

    @pl.when(sid < 15)
    def _():
        pltpu.sync_copy(t_hbm.at[pl.ds(sid * 640, 640)], t_sh.at[pl.ds(sid * 640, 640)])

    @pl.when(sid == 15)
    def _():
        pltpu.sync_copy(t_hbm.at[pl.ds(9600, 400)], t_sh.at[pl.ds(9600, 400)])

    plsc.subcore_barrier()

    ebase_w = wid * _GEPW
    nch = jnp.where(wid < 31, _GEPW // _GCH, (_E - 31 * _GEPW) // _GCH)
    gsems = (gsem0, gsem1)

    def fire(j, b):
        base = ebase_w + j * _GCH
        pltpu.sync_copy(ei0_hbm.at[pl.ds(base, _GCH)], idx0.at[b])
        pltpu.sync_copy(ei1_hbm.at[pl.ds(base, _GCH)], idx1.at[b])
        pltpu.async_copy(t_sh.at[idx0.at[b]], rt0.at[b], gsems[b])
        pltpu.async_copy(t_sh.at[idx1.at[b]], rt1.at[b], gsems[b])

    fire(0, 0)

    def pair(i, carry):
        for b in range(2):
            j = 2 * i + b
            nb = 1 - b

            @pl.when(j + 1 < nch)
            def _():
                fire(j + 1, nb)

            pltpu.make_async_copy(t_sh.at[idx0.at[b]], rt0.at[b], gsems[b]).wait()
            pltpu.make_async_copy(t_sh.at[idx1.at[b]], rt1.at[b], gsems[b]).wait()

            @pl.when(j >= 1)
            def _():
                pltpu.make_async_copy(
                    rs, gs_hbm.at[pl.ds(ebase_w + (j - 1) * _GCH, _GCH)],
                    wsem).wait()

            def row_add(r, c2):
                for cc in range(4):
                    rs[r, pl.ds(16 * cc, 16)] = (
                        rt0[b, r, pl.ds(16 * cc, 16)]
                        + rt1[b, r, pl.ds(_H + 16 * cc, 16)])
                return c2

            lax.fori_loop(0, _GCH, row_add, 0)
            pltpu.async_copy(rs,
                             gs_hbm.at[pl.ds(ebase_w + j * _GCH, _GCH)],
                             wsem)
        return carry

    lax.fori_loop(0, nch // 2, pair, 0)
    pltpu.make_async_copy(rs, gs_hbm.at[pl.ds(ebase_w + (nch - 1) * _GCH, _GCH)],
                          wsem).wait()


def _stage_sc_gather(t_tab, ei0, ei1):
    f = pl.kernel(
        _sc_gather_body,
        out_type=jax.ShapeDtypeStruct((_E, _H), jnp.float32),
        mesh=plsc.VectorSubcoreMesh(core_axis_name="c", subcore_axis_name="s"),
        scratch_types=[
            pltpu.VMEM_SHARED((_N, 2 * _H), jnp.float32),
            pltpu.VMEM((2, _GCH), jnp.int32),
            pltpu.VMEM((2, _GCH), jnp.int32),
            pltpu.VMEM((2, _GCH, 2 * _H), jnp.float32),
            pltpu.VMEM((2, _GCH, 2 * _H), jnp.float32),
            pltpu.VMEM((_GCH, _H), jnp.float32),
            pltpu.SemaphoreType.DMA,
            pltpu.SemaphoreType.DMA,
            pltpu.SemaphoreType.DMA,
        ],
    )
    return f(t_tab, ei0, ei1)


def _silu(x):
    return x / (1.0 + jnp.exp(-x))


def _ab_body(sx_ref, rx_ref, w1s_ref, w1r_ref, t_ref):
    t_ref[:, 0:_H] = jnp.dot(sx_ref[...], w1s_ref[...], preferred_element_type=jnp.float32)
    t_ref[:, _H:2 * _H] = jnp.dot(rx_ref[...], w1r_ref[...], preferred_element_type=jnp.float32)


def _stage_ab(sender_x, receiver_x, w1s, w1r):
    return pl.pallas_call(
        _ab_body,
        out_shape=jax.ShapeDtypeStruct((_N, 2 * _H), jnp.float32),
    )(sender_x, receiver_x, w1s, w1r)


def _edge_mlp_body(gs_ref, ea_ref, w1e_ref, b1_ref, w2_ref, b2_ref,
                   o_ref, stats_ref, sacc):
    pre = (gs_ref[...]
           + jnp.dot(ea_ref[...], w1e_ref[...], preferred_element_type=jnp.float32)
           + b1_ref[...])
    h = _silu(pre)
    o = jnp.dot(h, w2_ref[...], preferred_element_type=jnp.float32) + b2_ref[...]
    o_ref[...] = o
    i = pl.program_id(0)

    @pl.when(i == 0)
    def _():
        sacc[0] = 0.0
        sacc[1] = 0.0

    sacc[0] += jnp.sum(o)
    sacc[1] += jnp.sum(o * o)

    @pl.when(i == pl.num_programs(0) - 1)
    def _():
        stats_ref[0] = sacc[0]
        stats_ref[1] = sacc[1]


def _stage_edge_mlp(gs, edge_attr, w1e, eb1, eW2, eb2):
    return pl.pallas_call(
        _edge_mlp_body,
        grid=(_NB,),
        in_specs=[
            pl.BlockSpec((_EB, _H), lambda i: (i, 0)),
            pl.BlockSpec((_EB, _DE), lambda i: (i, 0)),
            pl.BlockSpec((_DE, _H), lambda i: (0, 0)),
            pl.BlockSpec((1, _H), lambda i: (0, 0)),
            pl.BlockSpec((_H, _DE), lambda i: (0, 0)),
            pl.BlockSpec((1, _DE), lambda i: (0, 0)),
        ],
        out_specs=[
            pl.BlockSpec((_EB, _DE), lambda i: (i, 0)),
            pl.BlockSpec(memory_space=pltpu.SMEM),
        ],
        out_shape=(
            jax.ShapeDtypeStruct((_E, _DE), jnp.float32),
            jax.ShapeDtypeStruct((2,), jnp.float32),
        ),
        scratch_shapes=[pltpu.SMEM((2,), jnp.float32)],
    )(gs, edge_attr, w1e, eb1, eW2, eb2)


_SCCH = 64
_SEPW = 10240


def _sc_scatter_body(o2_hbm, ei1_hbm, p_hbm,
                     acc, obuf, pay, idx_v, zbuf,
                     osem0, osem1, osem2, ssem0, ssem1, ssem2):
    sid = lax.axis_index("s")
    cid = lax.axis_index("c")
    wid = sid * _NC + cid
    osems = (osem0, osem1, osem2)
    ssems = (ssem0, ssem1, ssem2)

    def zrow(r, c2):
        for s in range(8):
            zbuf[r, pl.ds(16 * s, 16)] = jnp.zeros((16,), jnp.float32)
        return c2

    lax.fori_loop(0, 80, zrow, 0)

    def zcp(j, c2):
        pltpu.sync_copy(zbuf, acc.at[pl.ds(sid * 640 + j * 80, 80)])
        return c2

    @pl.when(sid < 15)
    def _():
        lax.fori_loop(0, 8, zcp, 0)

    @pl.when(sid == 15)
    def _():
        lax.fori_loop(0, 5, zcp, 0)

    ones0 = jnp.where(lax.iota(jnp.int32, 16) == 0,
                      jnp.float32(1.0), jnp.float32(0.0))

    def crow(r, c2):
        for s3 in range(3):
            pay[s3, r, pl.ds(16, 16)] = ones0
            for s in range(2, 8):
                pay[s3, r, pl.ds(16 * s, 16)] = jnp.zeros((16,), jnp.float32)
        return c2

    lax.fori_loop(0, _SCCH, crow, 0)

    plsc.subcore_barrier()

    ebase_w = wid * _SEPW
    rbase_w = wid * (_SEPW // 8)
    nch = jnp.where(wid < 31, _SEPW // _SCCH, (_E - 31 * _SEPW) // _SCCH)

    def fire(j, s):
        pltpu.sync_copy(ei1_hbm.at[pl.ds(ebase_w + j * _SCCH, _SCCH)], idx_v.at[s])
        pltpu.async_copy(
            o2_hbm.at[pl.ds(rbase_w + j * (_SCCH // 8), _SCCH // 8)],
            obuf.at[s], osems[s])

    def work(j, s):
        pltpu.make_async_copy(
            o2_hbm.at[pl.ds(rbase_w, _SCCH // 8)], obuf.at[s], osems[s]).wait()

        def prow(r, c3):
            for q in range(8):
                pay[s, 8 * r + q, pl.ds(0, 16)] = obuf[s, r, pl.ds(16 * q, 16)]
            return c3

        lax.fori_loop(0, _SCCH // 8, prow, 0)
        pltpu.async_copy(pay.at[s], acc.at[idx_v.at[s]], ssems[s], add=True)

    def drain_scatter(s):
        pltpu.make_async_copy(pay.at[s], acc.at[idx_v.at[s]], ssems[s]).wait()

    fire(0, 0)

    def triple(i, c2):
        for b in range(3):
            j = 3 * i + b
            nb = (b + 1) % 3

            @pl.when(j >= 2)
            def _():
                drain_scatter(nb)

            fire(j + 1, nb)
            work(j, b)
        return c2

    lax.fori_loop(0, (nch - 1) // 3, triple, 0)
    work(nch - 1, 0)
    drain_scatter(1)
    drain_scatter(2)
    drain_scatter(0)

    plsc.subcore_barrier()

    @pl.when(sid < 15)
    def _():
        pltpu.sync_copy(acc.at[pl.ds(sid * 640, 640)],
                        p_hbm.at[cid, pl.ds(sid * 640, 640)])

    @pl.when(sid == 15)
    def _():
        pltpu.sync_copy(acc.at[pl.ds(9600, 400)],
                        p_hbm.at[cid, pl.ds(9600, 400)])


def _stage_sc_scatter(o2, ei1):
    f = pl.kernel(
        _sc_scatter_body,
        out_type=jax.ShapeDtypeStruct((_NC, _N, 128), jnp.float32),
        mesh=plsc.VectorSubcoreMesh(core_axis_name="c", subcore_axis_name="s"),
        scratch_types=[
            pltpu.VMEM_SHARED((_N, 128), jnp.float32),
            pltpu.VMEM((3, _SCCH // 8, 128), jnp.float32),
            pltpu.VMEM((3, _SCCH, 128), jnp.float32),
            pltpu.VMEM((3, _SCCH), jnp.int32),
            pltpu.VMEM((80, 128), jnp.float32),
            pltpu.SemaphoreType.DMA,
            pltpu.SemaphoreType.DMA,
            pltpu.SemaphoreType.DMA,
            pltpu.SemaphoreType.DMA,
            pltpu.SemaphoreType.DMA,
            pltpu.SemaphoreType.DMA,
        ],
    )
    return f(o2, ei1)


def _edge_norm_body(stats_ref, o_ref, ea_ref, eg_ref, ebt_ref, out_ref):
    denom = 1.0 / (_E * _DE)
    mu = stats_ref[0] * denom
    var = stats_ref[1] * denom - mu * mu
    inv = lax.rsqrt(var + _EPS)
    a = eg_ref[...] * inv
    c = ebt_ref[...] - mu * a
    out_ref[...] = ea_ref[...] + o_ref[...] * a + c


def _stage_edge_norm(stats, o, edge_attr, eg, ebt):
    return pl.pallas_call(
        _edge_norm_body,
        grid=(_NB,),
        in_specs=[
            pl.BlockSpec(memory_space=pltpu.SMEM),
            pl.BlockSpec((_EB, _DE), lambda i: (i, 0)),
            pl.BlockSpec((_EB, _DE), lambda i: (i, 0)),
            pl.BlockSpec((1, _DE), lambda i: (0, 0)),
            pl.BlockSpec((1, _DE), lambda i: (0, 0)),
        ],
        out_specs=pl.BlockSpec((_EB, _DE), lambda i: (i, 0)),
        out_shape=jax.ShapeDtypeStruct((_E, _DE), jnp.float32),
    )(stats, o, edge_attr, eg, ebt)


def _node_body(stats_ref, p_ref, rx_ref, sx_ref,
               nW1r_ref, nW1e_ref, nb1_ref, nW2_ref, nb2_ref, ng_ref, nbt_ref,
               sW1_ref, sb1_ref, sW2_ref, sb2_ref, sg_ref, sbt_ref,
               eg_ref, ebt_ref,
               rout_ref, sout_ref):
    denom = 1.0 / (_E * _DE)
    mu = stats_ref[0] * denom
    var = stats_ref[1] * denom - mu * mu
    inv = lax.rsqrt(var + _EPS)
    a = eg_ref[...] * inv
    c = ebt_ref[...] - mu * a

    p = p_ref[...]
    seg = p[0:_N, 0:_DE] + p[_N:2 * _N, 0:_DE]
    cnt = p[0:_N, _DE:_DE + 1] + p[_N:2 * _N, _DE:_DE + 1]
    aggr = (seg * a + cnt * c) / jnp.maximum(cnt, 1.0)

    rx = rx_ref[...]
    pre = (jnp.dot(rx, nW1r_ref[...], preferred_element_type=jnp.float32)
           + jnp.dot(aggr, nW1e_ref[...], preferred_element_type=jnp.float32)
           + nb1_ref[...])
    h = _silu(pre)
    y = jnp.dot(h, nW2_ref[...], preferred_element_type=jnp.float32) + nb2_ref[...]
    mu_y = jnp.mean(y)
    var_y = jnp.mean((y - mu_y) ** 2)
    yn = (y - mu_y) * lax.rsqrt(var_y + _EPS) * ng_ref[...] + nbt_ref[...]
    rout_ref[...] = rx + yn

    sx = sx_ref[...]
    pre_s = jnp.dot(sx, sW1_ref[...], preferred_element_type=jnp.float32) + sb1_ref[...]
    hs = _silu(pre_s)
    ys = jnp.dot(hs, sW2_ref[...], preferred_element_type=jnp.float32) + sb2_ref[...]
    mu_s = jnp.mean(ys)
    var_s = jnp.mean((ys - mu_s) ** 2)
    ysn = (ys - mu_s) * lax.rsqrt(var_s + _EPS) * sg_ref[...] + sbt_ref[...]
    sout_ref[...] = sx + ysn


def _stage_node(stats, p, receiver_x, sender_x,
                nW1r, nW1e, nb1, nW2, nb2, ng, nbt,
                sW1, sb1, sW2, sb2, sg, sbt, eg, ebt):
    n_rest = 18
    return pl.pallas_call(
        _node_body,
        in_specs=[pl.BlockSpec(memory_space=pltpu.SMEM)]
                 + [pl.BlockSpec() for _ in range(n_rest)],
        out_shape=(
            jax.ShapeDtypeStruct((_N, _DS), jnp.float32),
            jax.ShapeDtypeStruct((_N, _DS), jnp.float32),
        ),
    )(stats, p, receiver_x, sender_x,
      nW1r, nW1e, nb1, nW2, nb2, ng, nbt,
      sW1, sb1, sW2, sb2, sg, sbt, eg, ebt)


def kernel(sender_x, receiver_x, edge_attr, edge_index,
           eW1, eb1, eW2, eb2, eg, ebt,
           nW1, nb1, nW2, nb2, ng, nbt,
           sW1, sb1, sW2, sb2, sg, sbt):
    ei0 = edge_index[0]
    ei1 = edge_index[1]

    w1s = eW1[0:_DS]
    w1r = eW1[_DS:2 * _DS]
    w1e = eW1[2 * _DS:]
    eb1_2 = eb1.reshape(1, _H)
    eb2_2 = eb2.reshape(1, _DE)
    eg_2 = eg.reshape(1, _DE)
    ebt_2 = ebt.reshape(1, _DE)
    nW1r = nW1[0:_DS]
    nW1e = nW1[_DS:]
    nb1_2 = nb1.reshape(1, _H)
    nb2_2 = nb2.reshape(1, _DS)
    ng_2 = ng.reshape(1, _DS)
    nbt_2 = nbt.reshape(1, _DS)
    sb1_2 = sb1.reshape(1, _H)
    sb2_2 = sb2.reshape(1, _DS)
    sg_2 = sg.reshape(1, _DS)
    sbt_2 = sbt.reshape(1, _DS)

    t_tab = _stage_ab(sender_x, receiver_x, w1s, w1r)

    gs = _stage_sc_gather(t_tab, ei0, ei1)

    o, stats = _stage_edge_mlp(gs, edge_attr, w1e, eb1_2, eW2, eb2_2)

    o2 = o.reshape(_E // 8, 128)
    p = _stage_sc_scatter(o2, ei1).reshape(2 * _N, 128)

    edge_out = _stage_edge_norm(stats, o, edge_attr, eg_2, ebt_2)

    receiver_out, sender_out = _stage_node(
        stats, p, receiver_x, sender_x,
        nW1r, nW1e, nb1_2, nW2, nb2_2, ng_2, nbt_2,
        sW1, sb1_2, sW2, sb2_2, sg_2, sbt_2, eg_2, ebt_2)

    return (sender_out, receiver_out, edge_out)
